# baseline (device time: 20419 ns/iter reference)
import jax
import jax.numpy as jnp
from jax import lax
from jax.experimental import pallas as pl
from jax.experimental.pallas import tpu as pltpu

N_DEV = 8


def kernel(x, Wq, Wo, K_ext, V_ext):
    B, Sq, D = x.shape
    Skv, Hkv, Dh = K_ext.shape[1:]
    d_local = Wq.shape[1]
    hq_local = d_local // Dh
    group = (hq_local * N_DEV) // Hkv
    kv_local = hq_local // group
    rows = B * Sq
    crows = rows // N_DEV
    cpb = Sq // crows

    my = lax.axis_index("i")

    K_loc = lax.dynamic_slice_in_dim(K_ext, my * kv_local, kv_local, axis=2)
    V_loc = lax.dynamic_slice_in_dim(V_ext, my * kv_local, kv_local, axis=2)
    K_loc = K_loc.reshape(B, Skv, kv_local * Dh)
    V_loc = V_loc.reshape(B, Skv, kv_local * Dh)

    def body(x_ref, wq_ref, wo_ref, k_ref, v_ref, out_ref,
             p_ref, rs_ref, r_ref, send_sems, rs_sems, ag_sems):
        my_pos = lax.axis_index("i")

        barrier_sem = pltpu.get_barrier_semaphore()
        for p in range(N_DEV):
            @pl.when(p != my_pos)
            def _():
                pl.semaphore_signal(
                    barrier_sem, inc=1,
                    device_id=(p,), device_id_type=pl.DeviceIdType.MESH,
                )
        pl.semaphore_wait(barrier_sem, N_DEV - 1)

        q = jnp.dot(x_ref[...].reshape(rows, D), wq_ref[...],
                    preferred_element_type=jnp.float32)

        rs_sends = []
        for b in range(B):
            o_blocks = [None] * hq_local
            for g in range(kv_local):
                qg = jnp.concatenate(
                    [q[b * Sq:(b + 1) * Sq,
                       (g * group + t) * Dh:(g * group + t + 1) * Dh]
                     for t in range(group)], axis=0)
                kh = k_ref[b, :, g * Dh:(g + 1) * Dh]
                vh = v_ref[b, :, g * Dh:(g + 1) * Dh]
                s = lax.dot_general(
                    qg, kh, (((1,), (1,)), ((), ())),
                    preferred_element_type=jnp.float32) * 0.125
                p = jnp.exp(s)
                l = jnp.sum(p, axis=1, keepdims=True)
                og = jnp.dot(p, vh, preferred_element_type=jnp.float32) / l
                for t in range(group):
                    o_blocks[g * group + t] = og[t * Sq:(t + 1) * Sq]
            O = jnp.concatenate(o_blocks, axis=1)
            p_ref[b * Sq:(b + 1) * Sq, :] = jnp.dot(
                O, wo_ref[...], preferred_element_type=jnp.float32)

            for j in range(b * cpb, (b + 1) * cpb):
                rdma = pltpu.make_async_remote_copy(
                    src_ref=p_ref.at[pl.ds(j * crows, crows), :],
                    dst_ref=rs_ref.at[my_pos],
                    send_sem=send_sems.at[j],
                    recv_sem=rs_sems.at[my_pos],
                    device_id=(j,),
                    device_id_type=pl.DeviceIdType.MESH,
                )
                rs_sends.append(rdma)

                @pl.when(j != my_pos)
                def _():
                    rdma.start()

        acc = p_ref[pl.ds(my_pos * crows, crows), :]
        for s in range(N_DEV):
            recv = pltpu.make_async_remote_copy(
                src_ref=p_ref.at[pl.ds(0, crows), :],
                dst_ref=rs_ref.at[s],
                send_sem=send_sems.at[s],
                recv_sem=rs_sems.at[s],
                device_id=(s,),
                device_id_type=pl.DeviceIdType.MESH,
            )

            @pl.when(s != my_pos)
            def _():
                recv.wait_recv()
            acc = acc + jnp.where(s == my_pos,
                                  jnp.zeros((crows, D), jnp.float32),
                                  rs_ref[s])
        r_ref[...] = acc

        for j in range(N_DEV):
            @pl.when(j != my_pos)
            def _():
                rs_sends[j].wait_send()

        my_b = my_pos // cpb
        my_r0 = lax.rem(my_pos, cpb) * crows
        ag_sends = []
        for j in range(N_DEV):
            rdma = pltpu.make_async_remote_copy(
                src_ref=r_ref,
                dst_ref=out_ref.at[my_b, pl.ds(my_r0, crows), :],
                send_sem=send_sems.at[j],
                recv_sem=ag_sems.at[my_pos],
                device_id=(j,),
                device_id_type=pl.DeviceIdType.MESH,
            )
            ag_sends.append(rdma)

            @pl.when(j != my_pos)
            def _():
                rdma.start()

        out_ref[my_b, pl.ds(my_r0, crows), :] = r_ref[...]

        for o in range(N_DEV):
            recv = pltpu.make_async_remote_copy(
                src_ref=r_ref,
                dst_ref=out_ref.at[o // cpb,
                                   pl.ds((o % cpb) * crows, crows), :],
                send_sem=send_sems.at[o],
                recv_sem=ag_sems.at[o],
                device_id=(o,),
                device_id_type=pl.DeviceIdType.MESH,
            )

            @pl.when(o != my_pos)
            def _():
                recv.wait_recv()
        for j in range(N_DEV):
            @pl.when(j != my_pos)
            def _():
                ag_sends[j].wait_send()

    return pl.pallas_call(
        body,
        out_shape=jax.ShapeDtypeStruct((B, Sq, D), jnp.float32),
        in_specs=[pl.BlockSpec(memory_space=pltpu.VMEM)] * 5,
        out_specs=pl.BlockSpec(memory_space=pltpu.VMEM),
        scratch_shapes=[
            pltpu.VMEM((rows, D), jnp.float32),
            pltpu.VMEM((N_DEV, crows, D), jnp.float32),
            pltpu.VMEM((crows, D), jnp.float32),
            pltpu.SemaphoreType.DMA((N_DEV,)),
            pltpu.SemaphoreType.DMA((N_DEV,)),
            pltpu.SemaphoreType.DMA((N_DEV,)),
        ],
        compiler_params=pltpu.CompilerParams(collective_id=0),
    )(x, Wq, Wo, K_loc, V_loc)


# device time: 16615 ns/iter; 1.2289x vs baseline; 1.2289x over previous
import jax
import jax.numpy as jnp
from jax import lax
from jax.experimental import pallas as pl
from jax.experimental.pallas import tpu as pltpu

N_DEV = 8


def kernel(x, Wq, Wo, K_ext, V_ext):
    B, Sq, D = x.shape
    Skv, Hkv, Dh = K_ext.shape[1:]
    d_local = Wq.shape[1]
    hq_local = d_local // Dh
    group = (hq_local * N_DEV) // Hkv
    kv_local = hq_local // group
    rows = B * Sq
    crows = rows // N_DEV
    cpb = Sq // crows

    my = lax.axis_index("i")

    K_loc = lax.dynamic_slice_in_dim(K_ext, my * kv_local, kv_local, axis=2)
    V_loc = lax.dynamic_slice_in_dim(V_ext, my * kv_local, kv_local, axis=2)
    K_loc = K_loc.reshape(B, Skv, kv_local * Dh)
    V_loc = V_loc.reshape(B, Skv, kv_local * Dh)
    x2d = x.reshape(rows, D)

    def body(x_ref, wq_ref, wo_ref, k_ref, v_ref, out_ref,
             p_ref, rs_ref, r_ref, ag_ref, send_sems, rs_sems, ag_sems):
        my_pos = lax.axis_index("i")

        barrier_sem = pltpu.get_barrier_semaphore()
        for p in range(N_DEV):
            @pl.when(p != my_pos)
            def _():
                pl.semaphore_signal(
                    barrier_sem, inc=1,
                    device_id=(p,), device_id_type=pl.DeviceIdType.MESH,
                )

        wq16 = wq_ref[...].astype(jnp.bfloat16)
        wo16 = wo_ref[...].astype(jnp.bfloat16)
        q = jnp.dot(x_ref[...].astype(jnp.bfloat16), wq16,
                    preferred_element_type=jnp.float32)

        o_all = []
        for b in range(B):
            o_blocks = [None] * hq_local
            for g in range(kv_local):
                qg = jnp.concatenate(
                    [q[b * Sq:(b + 1) * Sq,
                       (g * group + t) * Dh:(g * group + t + 1) * Dh]
                     for t in range(group)], axis=0)
                kh = k_ref[b, :, g * Dh:(g + 1) * Dh].astype(jnp.bfloat16)
                vh = v_ref[b, :, g * Dh:(g + 1) * Dh].astype(jnp.bfloat16)
                s = lax.dot_general(
                    qg.astype(jnp.bfloat16), kh, (((1,), (1,)), ((), ())),
                    preferred_element_type=jnp.float32) * 0.125
                p = jnp.exp(s)
                l = jnp.sum(p, axis=1, keepdims=True)
                og = jnp.dot(p.astype(jnp.bfloat16), vh,
                             preferred_element_type=jnp.float32) / l
                for t in range(group):
                    o_blocks[g * group + t] = og[t * Sq:(t + 1) * Sq]
            o_all.append(jnp.concatenate(o_blocks, axis=1))
        O = jnp.concatenate(o_all, axis=0)
        p_ref[...] = jnp.dot(O.astype(jnp.bfloat16), wo16,
                             preferred_element_type=jnp.float32
                             ).astype(jnp.bfloat16)

        pl.semaphore_wait(barrier_sem, N_DEV - 1)

        rs_sends = []
        for j in range(N_DEV):
            rdma = pltpu.make_async_remote_copy(
                src_ref=p_ref.at[pl.ds(j * crows, crows), :],
                dst_ref=rs_ref.at[my_pos],
                send_sem=send_sems.at[j],
                recv_sem=rs_sems.at[my_pos],
                device_id=(j,),
                device_id_type=pl.DeviceIdType.MESH,
            )
            rs_sends.append(rdma)

            @pl.when(j != my_pos)
            def _():
                rdma.start()

        acc = p_ref[pl.ds(my_pos * crows, crows), :].astype(jnp.float32)
        for s in range(N_DEV):
            recv = pltpu.make_async_remote_copy(
                src_ref=p_ref.at[pl.ds(0, crows), :],
                dst_ref=rs_ref.at[s],
                send_sem=send_sems.at[s],
                recv_sem=rs_sems.at[s],
                device_id=(s,),
                device_id_type=pl.DeviceIdType.MESH,
            )

            @pl.when(s != my_pos)
            def _():
                recv.wait_recv()
            acc = acc + jnp.where(s == my_pos,
                                  jnp.zeros((crows, D), jnp.float32),
                                  rs_ref[s].astype(jnp.float32))
        r_ref[...] = acc.astype(jnp.bfloat16)

        for j in range(N_DEV):
            @pl.when(j != my_pos)
            def _():
                rs_sends[j].wait_send()

        ag_sends = []
        for j in range(N_DEV):
            rdma = pltpu.make_async_remote_copy(
                src_ref=r_ref,
                dst_ref=ag_ref.at[my_pos],
                send_sem=send_sems.at[j],
                recv_sem=ag_sems.at[my_pos],
                device_id=(j,),
                device_id_type=pl.DeviceIdType.MESH,
            )
            ag_sends.append(rdma)

            @pl.when(j != my_pos)
            def _():
                rdma.start()

        for o in range(N_DEV):
            recv = pltpu.make_async_remote_copy(
                src_ref=r_ref,
                dst_ref=ag_ref.at[o],
                send_sem=send_sems.at[o],
                recv_sem=ag_sems.at[o],
                device_id=(o,),
                device_id_type=pl.DeviceIdType.MESH,
            )

            @pl.when(o != my_pos)
            def _():
                recv.wait_recv()
            chunk = jnp.where(o == my_pos, r_ref[...], ag_ref[o])
            out_ref[o // cpb, (o % cpb) * crows:(o % cpb + 1) * crows, :] = (
                chunk.astype(jnp.float32))

        for j in range(N_DEV):
            @pl.when(j != my_pos)
            def _():
                ag_sends[j].wait_send()

    return pl.pallas_call(
        body,
        out_shape=jax.ShapeDtypeStruct((B, Sq, D), jnp.float32),
        in_specs=[pl.BlockSpec(memory_space=pltpu.VMEM)] * 5,
        out_specs=pl.BlockSpec(memory_space=pltpu.VMEM),
        scratch_shapes=[
            pltpu.VMEM((rows, D), jnp.bfloat16),
            pltpu.VMEM((N_DEV, crows, D), jnp.bfloat16),
            pltpu.VMEM((crows, D), jnp.bfloat16),
            pltpu.VMEM((N_DEV, crows, D), jnp.bfloat16),
            pltpu.SemaphoreType.DMA((N_DEV,)),
            pltpu.SemaphoreType.DMA((N_DEV,)),
            pltpu.SemaphoreType.DMA((N_DEV,)),
        ],
        compiler_params=pltpu.CompilerParams(collective_id=0),
    )(x2d, Wq, Wo, K_loc, V_loc)


# device time: 15821 ns/iter; 1.2906x vs baseline; 1.0502x over previous
import jax
import jax.numpy as jnp
from jax import lax
from jax.experimental import pallas as pl
from jax.experimental.pallas import tpu as pltpu

N_DEV = 8


def kernel(x, Wq, Wo, K_ext, V_ext):
    B, Sq, D = x.shape
    Skv, Hkv, Dh = K_ext.shape[1:]
    d_local = Wq.shape[1]
    hq_local = d_local // Dh
    group = (hq_local * N_DEV) // Hkv
    kv_local = hq_local // group
    rows = B * Sq
    crows = rows // N_DEV
    cpb = Sq // crows

    my = lax.axis_index("i")

    K_loc = lax.dynamic_slice_in_dim(K_ext, my * kv_local, kv_local, axis=2)
    V_loc = lax.dynamic_slice_in_dim(V_ext, my * kv_local, kv_local, axis=2)
    K_loc = K_loc.reshape(B, Skv, kv_local * Dh).astype(jnp.bfloat16)
    V_loc = V_loc.reshape(B, Skv, kv_local * Dh).astype(jnp.bfloat16)
    x16 = x.reshape(rows, D).astype(jnp.bfloat16)
    Wq16 = Wq.astype(jnp.bfloat16)
    Wo16 = Wo.astype(jnp.bfloat16)

    def body(x_ref, wq_ref, wo_ref, k_ref, v_ref, out_ref,
             p_ref, rs_ref, r_ref, ag_ref,
             send_sems, ag_send_sems, rs_sems, ag_sems):
        my_pos = lax.axis_index("i")

        barrier_sem = pltpu.get_barrier_semaphore()
        for p in range(N_DEV):
            @pl.when(p != my_pos)
            def _():
                pl.semaphore_signal(
                    barrier_sem, inc=1,
                    device_id=(p,), device_id_type=pl.DeviceIdType.MESH,
                )

        q = jnp.dot(x_ref[...], wq_ref[...],
                    preferred_element_type=jnp.float32)

        o_all = []
        for b in range(B):
            o_blocks = [None] * hq_local
            for g in range(kv_local):
                qg = jnp.concatenate(
                    [q[b * Sq:(b + 1) * Sq,
                       (g * group + t) * Dh:(g * group + t + 1) * Dh]
                     for t in range(group)], axis=0)
                kh = k_ref[b, :, g * Dh:(g + 1) * Dh]
                vh = v_ref[b, :, g * Dh:(g + 1) * Dh]
                s = lax.dot_general(
                    qg.astype(jnp.bfloat16), kh, (((1,), (1,)), ((), ())),
                    preferred_element_type=jnp.float32) * 0.125
                p = jnp.exp(s)
                l = jnp.sum(p, axis=1, keepdims=True)
                og = jnp.dot(p.astype(jnp.bfloat16), vh,
                             preferred_element_type=jnp.float32) / l
                for t in range(group):
                    o_blocks[g * group + t] = og[t * Sq:(t + 1) * Sq]
            o_all.append(jnp.concatenate(o_blocks, axis=1))
        O = jnp.concatenate(o_all, axis=0)
        p_ref[...] = jnp.dot(O.astype(jnp.bfloat16), wo_ref[...],
                             preferred_element_type=jnp.float32
                             ).astype(jnp.bfloat16)

        pl.semaphore_wait(barrier_sem, N_DEV - 1)

        rs_sends = []
        for j in range(N_DEV):
            rdma = pltpu.make_async_remote_copy(
                src_ref=p_ref.at[pl.ds(j * crows, crows), :],
                dst_ref=rs_ref.at[my_pos],
                send_sem=send_sems.at[j],
                recv_sem=rs_sems.at[my_pos],
                device_id=(j,),
                device_id_type=pl.DeviceIdType.MESH,
            )
            rs_sends.append(rdma)

            @pl.when(j != my_pos)
            def _():
                rdma.start()

        acc = p_ref[pl.ds(my_pos * crows, crows), :].astype(jnp.float32)
        for s in range(N_DEV):
            recv = pltpu.make_async_remote_copy(
                src_ref=p_ref.at[pl.ds(0, crows), :],
                dst_ref=rs_ref.at[s],
                send_sem=send_sems.at[s],
                recv_sem=rs_sems.at[s],
                device_id=(s,),
                device_id_type=pl.DeviceIdType.MESH,
            )

            @pl.when(s != my_pos)
            def _():
                recv.wait_recv()
            acc = acc + jnp.where(s == my_pos,
                                  jnp.zeros((crows, D), jnp.float32),
                                  rs_ref[s].astype(jnp.float32))
        r_ref[...] = acc.astype(jnp.bfloat16)

        ag_sends = []
        for j in range(N_DEV):
            rdma = pltpu.make_async_remote_copy(
                src_ref=r_ref,
                dst_ref=ag_ref.at[my_pos],
                send_sem=ag_send_sems.at[j],
                recv_sem=ag_sems.at[my_pos],
                device_id=(j,),
                device_id_type=pl.DeviceIdType.MESH,
            )
            ag_sends.append(rdma)

            @pl.when(j != my_pos)
            def _():
                rdma.start()

        for o in range(N_DEV):
            recv = pltpu.make_async_remote_copy(
                src_ref=r_ref,
                dst_ref=ag_ref.at[o],
                send_sem=send_sems.at[o],
                recv_sem=ag_sems.at[o],
                device_id=(o,),
                device_id_type=pl.DeviceIdType.MESH,
            )

            @pl.when(o != my_pos)
            def _():
                recv.wait_recv()
            chunk = jnp.where(o == my_pos, r_ref[...], ag_ref[o])
            out_ref[o // cpb, (o % cpb) * crows:(o % cpb + 1) * crows, :] = (
                chunk.astype(jnp.float32))

        for j in range(N_DEV):
            @pl.when(j != my_pos)
            def _():
                rs_sends[j].wait_send()
                ag_sends[j].wait_send()

    return pl.pallas_call(
        body,
        out_shape=jax.ShapeDtypeStruct((B, Sq, D), jnp.float32),
        in_specs=[pl.BlockSpec(memory_space=pltpu.VMEM)] * 5,
        out_specs=pl.BlockSpec(memory_space=pltpu.VMEM),
        scratch_shapes=[
            pltpu.VMEM((rows, D), jnp.bfloat16),
            pltpu.VMEM((N_DEV, crows, D), jnp.bfloat16),
            pltpu.VMEM((crows, D), jnp.bfloat16),
            pltpu.VMEM((N_DEV, crows, D), jnp.bfloat16),
            pltpu.SemaphoreType.DMA((N_DEV,)),
            pltpu.SemaphoreType.DMA((N_DEV,)),
            pltpu.SemaphoreType.DMA((N_DEV,)),
            pltpu.SemaphoreType.DMA((N_DEV,)),
        ],
        compiler_params=pltpu.CompilerParams(collective_id=0),
    )(x16, Wq16, Wo16, K_loc, V_loc)
